# baseline (speedup 1.0000x reference)
import functools

import jax
import jax.numpy as jnp
from jax import lax
from jax.experimental import pallas as pl
from jax.experimental.pallas import tpu as pltpu
from jax.experimental.pallas import tpu_sc as plsc

NUM_NUMERIC = 64
EMB = 64
BATCH = 4096
FEAT = 128

_NC = 2
_NS = 16
_NW = _NC * _NS
_BPW = BATCH // _NW


def _sc_gather_body(uid_hbm, pid_hbm, utab_hbm, ptab_hbm, uout_hbm, pout_hbm,
                    uidx, pidx, urows, prows, usem, psem):
  wid = lax.axis_index("s") * _NC + lax.axis_index("c")
  base = wid * _BPW
  pltpu.sync_copy(uid_hbm.at[pl.ds(base, _BPW)], uidx)
  ucopy = pltpu.async_copy(utab_hbm.at[uidx], urows, usem)
  pltpu.sync_copy(pid_hbm.at[pl.ds(base, _BPW)], pidx)
  pcopy = pltpu.async_copy(ptab_hbm.at[pidx], prows, psem)
  ucopy.wait()
  pltpu.sync_copy(urows, uout_hbm.at[pl.ds(base, _BPW)])
  pcopy.wait()
  pltpu.sync_copy(prows, pout_hbm.at[pl.ds(base, _BPW)])


@jax.jit
def _sc_gather(uid, pid, utab, ptab):
  mesh = plsc.VectorSubcoreMesh(core_axis_name="c", subcore_axis_name="s")
  return pl.kernel(
      _sc_gather_body,
      mesh=mesh,
      compiler_params=pltpu.CompilerParams(use_tc_tiling_on_sc=False),
      out_type=[
          jax.ShapeDtypeStruct((BATCH, EMB), jnp.float32),
          jax.ShapeDtypeStruct((BATCH, EMB), jnp.float32),
      ],
      scratch_types=[
          pltpu.VMEM((_BPW,), jnp.int32),
          pltpu.VMEM((_BPW,), jnp.int32),
          pltpu.VMEM((_BPW, EMB), jnp.float32),
          pltpu.VMEM((_BPW, EMB), jnp.float32),
          pltpu.SemaphoreType.DMA,
          pltpu.SemaphoreType.DMA,
      ],
  )(uid, pid, utab, ptab)


def _mlp_body(uvec, pvec, ff, wnum, bnum, wsty, bsty, w1, b1, w2, b2, w3, b3,
              w4, b4, out):
  f32 = jnp.float32
  numeric = jnp.maximum(
      jnp.dot(ff[:, :NUM_NUMERIC], wnum[:], preferred_element_type=f32)
      + bnum[:], 0.0)
  style = jnp.maximum(
      jnp.dot(ff[:, NUM_NUMERIC:], wsty[:], preferred_element_type=f32)
      + bsty[:], 0.0)
  h = (jnp.dot(uvec[:], w1[0:EMB], preferred_element_type=f32)
       + jnp.dot(pvec[:], w1[EMB:2 * EMB], preferred_element_type=f32)
       + jnp.dot(numeric, w1[2 * EMB:3 * EMB], preferred_element_type=f32)
       + jnp.dot(style, w1[3 * EMB:4 * EMB], preferred_element_type=f32)
       + b1[:])
  h = jnp.maximum(h, 0.0)
  h = jnp.maximum(jnp.dot(h, w2[:], preferred_element_type=f32) + b2[:], 0.0)
  h = jnp.maximum(jnp.dot(h, w3[:], preferred_element_type=f32) + b3[:], 0.0)
  logit = jnp.sum(h * w4[:], axis=1, keepdims=True) + b4[:]
  out[:] = 1.0 / (1.0 + jnp.exp(-logit))


@jax.jit
def _mlp(uvec, pvec, ff, wnum, bnum, wsty, bsty, w1, b1, w2, b2, w3, b3, w4,
         b4):
  nblk = 4
  blk = BATCH // nblk
  row_spec = lambda width: pl.BlockSpec((blk, width), lambda i: (i, 0))
  full = lambda a: pl.BlockSpec(a.shape, lambda i: tuple(0 for _ in a.shape))
  return pl.pallas_call(
      _mlp_body,
      grid=(nblk,),
      in_specs=[
          row_spec(EMB),
          row_spec(EMB),
          row_spec(FEAT),
          full(wnum), full(bnum), full(wsty), full(bsty),
          full(w1), full(b1), full(w2), full(b2), full(w3), full(b3),
          full(w4), full(b4),
      ],
      out_specs=pl.BlockSpec((blk, 1), lambda i: (i, 0)),
      out_shape=jax.ShapeDtypeStruct((BATCH, 1), jnp.float32),
  )(uvec, pvec, ff, wnum, bnum, wsty, bsty, w1, b1, w2, b2, w3, b3, w4, b4)


def kernel(user_id, product_id, full_features, user_table, product_table,
           W_num, b_num, W_style, b_style, W1, b1, W2, b2, W3, b3, W4, b4):
  uid = user_id.astype(jnp.int32)
  pid = product_id.astype(jnp.int32)
  uvec, pvec = _sc_gather(uid, pid, user_table, product_table)
  return _mlp(uvec, pvec, full_features,
              W_num, b_num.reshape(1, EMB), W_style, b_style.reshape(1, EMB),
              W1, b1.reshape(1, 128), W2, b2.reshape(1, 64),
              W3, b3.reshape(1, 32), W4.reshape(1, 32), b4.reshape(1, 1))


# speedup vs baseline: 1.4515x; 1.4515x over previous
import functools

import jax
import jax.numpy as jnp
from jax import lax
from jax.experimental import pallas as pl
from jax.experimental.pallas import tpu as pltpu
from jax.experimental.pallas import tpu_sc as plsc

NUM_NUMERIC = 64
EMB = 64
BATCH = 4096
FEAT = 128

_NC = 2
_NS = 16
_NW = _NC * _NS
_BPW = BATCH // _NW


def _sc_gather_body(uid_hbm, pid_hbm, utab_hbm, ptab_hbm, uout_hbm, pout_hbm,
                    uidx_v, pidx_v, urows, prows, usem, psem):
  wid = lax.axis_index("s") * _NC + lax.axis_index("c")
  base = wid * _BPW
  pltpu.sync_copy(uid_hbm.at[pl.ds(base, _BPW)], uidx_v)
  pltpu.sync_copy(pid_hbm.at[pl.ds(base, _BPW)], pidx_v)
  lane = lax.iota(jnp.int32, 16)

  def issue(c, _):
    vu = uidx_v[pl.ds(c * 16, 16)]
    vp = pidx_v[pl.ds(c * 16, 16)]
    for l in range(16):
      r = jnp.sum(jnp.where(lane == l, vu, 0))
      pltpu.async_copy(utab_hbm.at[pl.ds(r, 1)],
                       urows.at[pl.ds(c * 16 + l, 1)], usem)
      q = jnp.sum(jnp.where(lane == l, vp, 0))
      pltpu.async_copy(ptab_hbm.at[pl.ds(q, 1)],
                       prows.at[pl.ds(c * 16 + l, 1)], psem)
    return 0

  lax.fori_loop(0, _BPW // 16, issue, 0)
  pltpu.make_async_copy(utab_hbm.at[pl.ds(0, _BPW)], urows, usem).wait()
  pltpu.sync_copy(urows, uout_hbm.at[pl.ds(base, _BPW)])
  pltpu.make_async_copy(ptab_hbm.at[pl.ds(0, _BPW)], prows, psem).wait()
  pltpu.sync_copy(prows, pout_hbm.at[pl.ds(base, _BPW)])


@jax.jit
def _sc_gather(uid, pid, utab, ptab):
  mesh = plsc.VectorSubcoreMesh(core_axis_name="c", subcore_axis_name="s")
  return pl.kernel(
      _sc_gather_body,
      mesh=mesh,
      compiler_params=pltpu.CompilerParams(needs_layout_passes=False),
      out_type=[
          jax.ShapeDtypeStruct((BATCH, EMB), jnp.float32),
          jax.ShapeDtypeStruct((BATCH, EMB), jnp.float32),
      ],
      scratch_types=[
          pltpu.VMEM((_BPW,), jnp.int32),
          pltpu.VMEM((_BPW,), jnp.int32),
          pltpu.VMEM((_BPW, EMB), jnp.float32),
          pltpu.VMEM((_BPW, EMB), jnp.float32),
          pltpu.SemaphoreType.DMA,
          pltpu.SemaphoreType.DMA,
      ],
  )(uid, pid, utab, ptab)


def _mlp_body(uvec, pvec, ff, wnum, bnum, wsty, bsty, w1, b1, w2, b2, w3, b3,
              w4, b4, out):
  f32 = jnp.float32
  numeric = jnp.maximum(
      jnp.dot(ff[:, :NUM_NUMERIC], wnum[:], preferred_element_type=f32)
      + bnum[:], 0.0)
  style = jnp.maximum(
      jnp.dot(ff[:, NUM_NUMERIC:], wsty[:], preferred_element_type=f32)
      + bsty[:], 0.0)
  h = (jnp.dot(uvec[:], w1[0:EMB], preferred_element_type=f32)
       + jnp.dot(pvec[:], w1[EMB:2 * EMB], preferred_element_type=f32)
       + jnp.dot(numeric, w1[2 * EMB:3 * EMB], preferred_element_type=f32)
       + jnp.dot(style, w1[3 * EMB:4 * EMB], preferred_element_type=f32)
       + b1[:])
  h = jnp.maximum(h, 0.0)
  h = jnp.maximum(jnp.dot(h, w2[:], preferred_element_type=f32) + b2[:], 0.0)
  h = jnp.maximum(jnp.dot(h, w3[:], preferred_element_type=f32) + b3[:], 0.0)
  logit = jnp.sum(h * w4[:], axis=1, keepdims=True) + b4[:]
  out[:] = 1.0 / (1.0 + jnp.exp(-logit))


@jax.jit
def _mlp(uvec, pvec, ff, wnum, bnum, wsty, bsty, w1, b1, w2, b2, w3, b3, w4,
         b4):
  nblk = 4
  blk = BATCH // nblk
  row_spec = lambda width: pl.BlockSpec((blk, width), lambda i: (i, 0))
  full = lambda a: pl.BlockSpec(a.shape, lambda i: tuple(0 for _ in a.shape))
  return pl.pallas_call(
      _mlp_body,
      grid=(nblk,),
      in_specs=[
          row_spec(EMB),
          row_spec(EMB),
          row_spec(FEAT),
          full(wnum), full(bnum), full(wsty), full(bsty),
          full(w1), full(b1), full(w2), full(b2), full(w3), full(b3),
          full(w4), full(b4),
      ],
      out_specs=pl.BlockSpec((blk, 1), lambda i: (i, 0)),
      out_shape=jax.ShapeDtypeStruct((BATCH, 1), jnp.float32),
  )(uvec, pvec, ff, wnum, bnum, wsty, bsty, w1, b1, w2, b2, w3, b3, w4, b4)


def kernel(user_id, product_id, full_features, user_table, product_table,
           W_num, b_num, W_style, b_style, W1, b1, W2, b2, W3, b3, W4, b4):
  uid = user_id.astype(jnp.int32)
  pid = product_id.astype(jnp.int32)
  uvec, pvec = _sc_gather(uid, pid, user_table, product_table)
  return _mlp(uvec, pvec, full_features,
              W_num, b_num.reshape(1, EMB), W_style, b_style.reshape(1, EMB),
              W1, b1.reshape(1, 128), W2, b2.reshape(1, 64),
              W3, b3.reshape(1, 32), W4.reshape(1, 32), b4.reshape(1, 1))
